# baseline (speedup 1.0000x reference)
import jax
import jax.numpy as jnp
from jax.experimental import pallas as pl
from jax.experimental.pallas import tpu as pltpu


_BS = 512


def _add_kernel(x_ref, emb_hbm, o_ref, emb_v, sems):
    s = pl.program_id(0)
    b = pl.program_id(1)
    n_seq = pl.num_programs(0)
    bs = emb_v.shape[1]
    slot = jax.lax.rem(s, 2)

    @pl.when(jnp.logical_and(s == 0, b == 0))
    def _prime():
        pltpu.make_async_copy(
            emb_hbm.at[pl.ds(0, bs)], emb_v.at[0], sems.at[0]
        ).start()

    @pl.when(jnp.logical_and(b == 0, s + 1 < n_seq))
    def _prefetch():
        nxt = jax.lax.rem(s + 1, 2)
        pltpu.make_async_copy(
            emb_hbm.at[pl.ds((s + 1) * bs, bs)], emb_v.at[nxt], sems.at[nxt]
        ).start()

    @pl.when(b == 0)
    def _wait():
        pltpu.make_async_copy(
            emb_hbm.at[pl.ds(0, bs)], emb_v.at[slot], sems.at[slot]
        ).wait()

    o_ref[...] = x_ref[...] + emb_v[slot][None, :, :]


def kernel(x, embedding):
    batch, seq_len, d_model = x.shape
    bs = _BS if seq_len % _BS == 0 else seq_len
    grid = (seq_len // bs, batch)
    return pl.pallas_call(
        _add_kernel,
        grid=grid,
        in_specs=[
            pl.BlockSpec((1, bs, d_model), lambda s, b: (b, s, 0)),
            pl.BlockSpec(memory_space=pltpu.MemorySpace.HBM),
        ],
        out_specs=pl.BlockSpec((1, bs, d_model), lambda s, b: (b, s, 0)),
        out_shape=jax.ShapeDtypeStruct((batch, seq_len, d_model), x.dtype),
        scratch_shapes=[
            pltpu.VMEM((2, bs, d_model), jnp.float32),
            pltpu.SemaphoreType.DMA((2,)),
        ],
    )(x, embedding)


# speedup vs baseline: 1.1113x; 1.1113x over previous
import jax
import jax.numpy as jnp
from jax.experimental import pallas as pl
from jax.experimental.pallas import tpu as pltpu


_BS = 2048


def _add_kernel(x_ref, emb_hbm, o_ref, emb_v, sems):
    s = pl.program_id(0)
    b = pl.program_id(1)
    n_seq = pl.num_programs(0)
    bs = emb_v.shape[1]
    slot = jax.lax.rem(s, 2)

    @pl.when(jnp.logical_and(s == 0, b == 0))
    def _prime():
        pltpu.make_async_copy(
            emb_hbm.at[pl.ds(0, bs)], emb_v.at[0], sems.at[0]
        ).start()

    @pl.when(jnp.logical_and(b == 0, s + 1 < n_seq))
    def _prefetch():
        nxt = jax.lax.rem(s + 1, 2)
        pltpu.make_async_copy(
            emb_hbm.at[pl.ds((s + 1) * bs, bs)], emb_v.at[nxt], sems.at[nxt]
        ).start()

    @pl.when(b == 0)
    def _wait():
        pltpu.make_async_copy(
            emb_hbm.at[pl.ds(0, bs)], emb_v.at[slot], sems.at[slot]
        ).wait()

    o_ref[...] = x_ref[...] + emb_v[slot][None, :, :]


def kernel(x, embedding):
    batch, seq_len, d_model = x.shape
    bs = _BS if seq_len % _BS == 0 else seq_len
    grid = (seq_len // bs, batch)
    return pl.pallas_call(
        _add_kernel,
        grid=grid,
        in_specs=[
            pl.BlockSpec((1, bs, d_model), lambda s, b: (b, s, 0)),
            pl.BlockSpec(memory_space=pltpu.MemorySpace.HBM),
        ],
        out_specs=pl.BlockSpec((1, bs, d_model), lambda s, b: (b, s, 0)),
        out_shape=jax.ShapeDtypeStruct((batch, seq_len, d_model), x.dtype),
        scratch_shapes=[
            pltpu.VMEM((2, bs, d_model), jnp.float32),
            pltpu.SemaphoreType.DMA((2,)),
        ],
    )(x, embedding)
